# Initial kernel scaffold; baseline (speedup 1.0000x reference)
import functools
import numpy as np
import jax
import jax.numpy as jnp
from jax import lax
from jax.experimental import pallas as pl
from jax.experimental.pallas import tpu as pltpu
from jax.experimental.pallas import tpu_sc as plsc

D_MODEL = 512
SEQ_LEN = 380
SEQ_PAD = 384
BATCH = 1024

NW = 32
BPW = BATCH // NW
CS = 96
NCHUNK = SEQ_PAD // CS
LANES = 16
KPR = D_MODEL // LANES


def _pe_matrix_np(d_model, seq_len):
    a, b = np.meshgrid(np.arange(d_model), np.arange(seq_len))
    pe_mat = b / 10000 ** (2 * (a // 2) / d_model)
    pe_mat[:, 0::2] = np.sin(pe_mat[:, 0::2])
    pe_mat[:, 1::2] = np.cos(pe_mat[:, 1::2])
    return pe_mat.astype(np.float32)


@functools.partial(
    pl.kernel,
    out_type=jax.ShapeDtypeStruct((BATCH * SEQ_LEN, D_MODEL), jnp.float32),
    mesh=plsc.VectorSubcoreMesh(core_axis_name="c", subcore_axis_name="s"),
    scratch_types=[
        pltpu.VMEM((BPW * SEQ_PAD,), jnp.int32),
        pltpu.VMEM((CS, D_MODEL), jnp.float32),
        pltpu.VMEM((CS, D_MODEL), jnp.float32),
        pltpu.SemaphoreType.DMA,
    ],
)
def _emb_pe_kernel(xpad_hbm, table_hbm, pe_hbm, out_hbm, idx_v, pe_v, rows_v, sem):
    wid = lax.axis_index("s") * 2 + lax.axis_index("c")
    pltpu.sync_copy(xpad_hbm.at[pl.ds(wid * (BPW * SEQ_PAD), BPW * SEQ_PAD)], idx_v)

    for cp in range(NCHUNK):
        vcnt = min(CS, SEQ_LEN - cp * CS)
        pltpu.sync_copy(pe_hbm.at[pl.ds(cp * CS, CS), :], pe_v)

        def seq_body(seq, carry, cp=cp, vcnt=vcnt):
            ioff = seq * SEQ_PAD + cp * CS
            pltpu.async_copy(
                table_hbm.at[idx_v.at[pl.ds(ioff, CS)]], rows_v, sem
            ).wait()

            def add_row(r, c):
                for k in range(KPR):
                    sl = pl.ds(k * LANES, LANES)
                    rows_v[r, sl] = rows_v[r, sl] + pe_v[r, sl]
                return c

            lax.fori_loop(0, vcnt, add_row, None)
            orow = (wid * BPW + seq) * SEQ_LEN + cp * CS
            pltpu.sync_copy(
                rows_v.at[pl.ds(0, vcnt), :], out_hbm.at[pl.ds(orow, vcnt), :]
            )
            return carry

        lax.fori_loop(0, BPW, seq_body, None)


def kernel(x, emb_table):
    pe = jnp.asarray(_pe_matrix_np(D_MODEL, SEQ_LEN))
    xpad = jnp.pad(x.astype(jnp.int32), ((0, 0), (0, SEQ_PAD - SEQ_LEN)))
    out = _emb_pe_kernel(xpad.reshape(-1), emb_table, pe)
    return out.reshape(BATCH, SEQ_LEN, D_MODEL)


# speedup vs baseline: 1.2501x; 1.2501x over previous
import functools
import numpy as np
import jax
import jax.numpy as jnp
from jax import lax
from jax.experimental import pallas as pl
from jax.experimental.pallas import tpu as pltpu
from jax.experimental.pallas import tpu_sc as plsc

D_MODEL = 512
SEQ_LEN = 380
BATCH = 1024
NROWS = BATCH * SEQ_LEN

NW = 32
RPW = NROWS // NW
F = 64
CPW = RPW // F
LANES = 16
KPR = D_MODEL // LANES


def _pe_matrix_np(d_model, seq_len):
    a, b = np.meshgrid(np.arange(d_model), np.arange(seq_len))
    pe_mat = b / 10000 ** (2 * (a // 2) / d_model)
    pe_mat[:, 0::2] = np.sin(pe_mat[:, 0::2])
    pe_mat[:, 1::2] = np.cos(pe_mat[:, 1::2])
    return pe_mat.astype(np.float32)


@functools.partial(
    pl.kernel,
    out_type=jax.ShapeDtypeStruct((NROWS, D_MODEL), jnp.float32),
    mesh=plsc.VectorSubcoreMesh(core_axis_name="c", subcore_axis_name="s"),
    scratch_types=[
        pltpu.VMEM((CPW, F), jnp.int32),
        pltpu.VMEM((CPW, F), jnp.int32),
        pltpu.VMEM((F, D_MODEL), jnp.float32),
        pltpu.VMEM((F, D_MODEL), jnp.float32),
        pltpu.SemaphoreType.DMA,
        pltpu.SemaphoreType.DMA,
    ],
)
def _emb_pe_kernel(x_hbm, table_hbm, pe_hbm, peidx_hbm, out_hbm,
                   idx_v, pidx_v, rows_v, pe_v, sem_a, sem_b):
    wid = lax.axis_index("s") * 2 + lax.axis_index("c")
    pltpu.sync_copy(x_hbm.at[wid], idx_v)
    pltpu.sync_copy(peidx_hbm.at[wid], pidx_v)

    def chunk_body(c, carry):
        cp_a = pltpu.async_copy(table_hbm.at[idx_v.at[c]], rows_v, sem_a)
        cp_b = pltpu.async_copy(pe_hbm.at[pidx_v.at[c]], pe_v, sem_b)
        cp_a.wait()
        cp_b.wait()

        def add_row(r, cc):
            for k in range(KPR):
                sl = pl.ds(k * LANES, LANES)
                rows_v[r, sl] = rows_v[r, sl] + pe_v[r, sl]
            return cc

        lax.fori_loop(0, F, add_row, None)
        pltpu.sync_copy(rows_v, out_hbm.at[pl.ds((wid * CPW + c) * F, F), :])
        return carry

    lax.fori_loop(0, CPW, chunk_body, None)


def kernel(x, emb_table):
    pe = jnp.asarray(_pe_matrix_np(D_MODEL, SEQ_LEN))
    xi = x.astype(jnp.int32).reshape(NW, CPW, F)
    pe_idx = jnp.tile(jnp.arange(SEQ_LEN, dtype=jnp.int32), BATCH)
    out = _emb_pe_kernel(xi, emb_table, pe, pe_idx.reshape(NW, CPW, F))
    return out.reshape(BATCH, SEQ_LEN, D_MODEL)


# speedup vs baseline: 1.4489x; 1.1591x over previous
import functools
import numpy as np
import jax
import jax.numpy as jnp
from jax import lax
from jax.experimental import pallas as pl
from jax.experimental.pallas import tpu as pltpu
from jax.experimental.pallas import tpu_sc as plsc

D_MODEL = 512
SEQ_LEN = 380
BATCH = 1024
NROWS = BATCH * SEQ_LEN

NW = 32
RPW = NROWS // NW
F = 32
CPW = RPW // F
LANES = 16
KPR = D_MODEL // LANES


def _pe_matrix_np(d_model, seq_len):
    a, b = np.meshgrid(np.arange(d_model), np.arange(seq_len))
    pe_mat = b / 10000 ** (2 * (a // 2) / d_model)
    pe_mat[:, 0::2] = np.sin(pe_mat[:, 0::2])
    pe_mat[:, 1::2] = np.cos(pe_mat[:, 1::2])
    return pe_mat.astype(np.float32)


@functools.partial(
    pl.kernel,
    out_type=jax.ShapeDtypeStruct((NROWS, D_MODEL), jnp.float32),
    mesh=plsc.VectorSubcoreMesh(core_axis_name="c", subcore_axis_name="s"),
    scratch_types=[
        pltpu.VMEM((CPW, F), jnp.int32),
        pltpu.VMEM((F, D_MODEL), jnp.float32),
        pltpu.VMEM((F, D_MODEL), jnp.float32),
        pltpu.VMEM((F, D_MODEL), jnp.float32),
        pltpu.VMEM((F, D_MODEL), jnp.float32),
        pltpu.VMEM((F,), jnp.int32),
        pltpu.VMEM((F,), jnp.int32),
        pltpu.SemaphoreType.DMA,
        pltpu.SemaphoreType.DMA,
        pltpu.SemaphoreType.DMA,
        pltpu.SemaphoreType.DMA,
    ],
)
def _emb_pe_kernel(x_hbm, table_hbm, pe_hbm, out_hbm,
                   idx_v, rows0, rows1, pev0, pev1, pidx0, pidx1,
                   gsem0, gsem1, ssem0, ssem1):
    rows = (rows0, rows1)
    pev = (pev0, pev1)
    pidx = (pidx0, pidx1)
    gsem = (gsem0, gsem1)
    ssem = (ssem0, ssem1)

    wid = lax.axis_index("s") * 2 + lax.axis_index("c")
    pltpu.sync_copy(x_hbm.at[wid], idx_v)

    def issue_gathers(c, b):
        base = lax.rem(c * F, SEQ_LEN)
        for h in range(F // LANES):
            vec = lax.rem(
                base + h * LANES + lax.iota(jnp.int32, LANES), SEQ_LEN
            )
            pidx[b][pl.ds(h * LANES, LANES)] = vec
        pltpu.async_copy(table_hbm.at[idx_v.at[c]], rows[b], gsem[b])
        pltpu.async_copy(pe_hbm.at[pidx[b]], pev[b], gsem[b])

    def wait_gathers(c, b):
        pltpu.make_async_copy(table_hbm.at[idx_v.at[c]], rows[b], gsem[b]).wait()
        pltpu.make_async_copy(pe_hbm.at[pidx[b]], pev[b], gsem[b]).wait()

    def wait_scatter(b):
        pltpu.make_async_copy(rows[b], out_hbm.at[pl.ds(0, F), :], ssem[b]).wait()

    issue_gathers(0, 0)

    def outer(c0, carry):
        for b in range(2):
            c = c0 + b
            ob = 1 - b

            def prefetch(c=c, b=b, ob=ob):
                if b == 0:
                    @pl.when(c >= 1)
                    def _():
                        wait_scatter(ob)
                else:
                    wait_scatter(ob)
                issue_gathers(c + 1, ob)

            pl.when(c + 1 < CPW)(prefetch)
            wait_gathers(c, b)

            def add_row(r, cc, b=b):
                for k in range(KPR):
                    sl = pl.ds(k * LANES, LANES)
                    rows[b][r, sl] = rows[b][r, sl] + pev[b][r, sl]
                return cc

            lax.fori_loop(0, F, add_row, None)
            pltpu.async_copy(
                rows[b], out_hbm.at[pl.ds((wid * CPW + c) * F, F), :], ssem[b]
            )
        return carry

    lax.fori_loop(0, CPW // 2, lambda i, cy: outer(i * 2, cy), None)
    wait_scatter(0)
    wait_scatter(1)


def kernel(x, emb_table):
    pe = jnp.asarray(_pe_matrix_np(D_MODEL, SEQ_LEN))
    xi = x.astype(jnp.int32).reshape(NW, CPW, F)
    out = _emb_pe_kernel(xi, emb_table, pe)
    return out.reshape(BATCH, SEQ_LEN, D_MODEL)


# speedup vs baseline: 1.9259x; 1.3292x over previous
import functools
import numpy as np
import jax
import jax.numpy as jnp
from jax import lax
from jax.experimental import pallas as pl
from jax.experimental.pallas import tpu as pltpu
from jax.experimental.pallas import tpu_sc as plsc

D_MODEL = 512
SEQ_LEN = 380
BATCH = 1024

NW = 32
BPW = BATCH // NW
CS = 32
NFULL = SEQ_LEN // CS
TAIL = SEQ_LEN - NFULL * CS
SEQ_PAD = (NFULL + 1) * CS
LANES = 16
KPR = D_MODEL // LANES


def _pe_matrix_np(d_model, seq_len):
    a, b = np.meshgrid(np.arange(d_model), np.arange(seq_len))
    pe_mat = b / 10000 ** (2 * (a // 2) / d_model)
    pe_mat[:, 0::2] = np.sin(pe_mat[:, 0::2])
    pe_mat[:, 1::2] = np.cos(pe_mat[:, 1::2])
    return pe_mat.astype(np.float32)


@functools.partial(
    pl.kernel,
    out_type=jax.ShapeDtypeStruct((BATCH, SEQ_LEN, D_MODEL), jnp.float32),
    mesh=plsc.VectorSubcoreMesh(core_axis_name="c", subcore_axis_name="s"),
    scratch_types=[
        pltpu.VMEM((BPW * (NFULL + 1), CS), jnp.int32),
        pltpu.VMEM((BPW, CS), jnp.int32),
        pltpu.VMEM((CS, D_MODEL), jnp.float32),
        pltpu.VMEM((CS, D_MODEL), jnp.float32),
        pltpu.VMEM((CS, D_MODEL), jnp.float32),
        pltpu.VMEM((TAIL, D_MODEL), jnp.float32),
        pltpu.SemaphoreType.DMA,
        pltpu.SemaphoreType.DMA,
        pltpu.SemaphoreType.DMA,
        pltpu.SemaphoreType.DMA,
    ],
)
def _emb_pe_kernel(xc_hbm, xt_hbm, table_hbm, pe_hbm, out_hbm,
                   idx_v, tidx_v, pe_v, rows0, rows1, tail_v,
                   gsem0, gsem1, ssem0, ssem1):
    rows = (rows0, rows1)
    gsem = (gsem0, gsem1)
    ssem = (ssem0, ssem1)

    wid = lax.axis_index("s") * 2 + lax.axis_index("c")
    pltpu.sync_copy(xc_hbm.at[wid], idx_v)
    pltpu.sync_copy(xt_hbm.at[wid], tidx_v)

    for cp in range(NFULL):
        pltpu.sync_copy(pe_hbm.at[pl.ds(cp * CS, CS), :], pe_v)

        def issue_gather(seq, b, cp=cp):
            pltpu.async_copy(
                table_hbm.at[idx_v.at[seq * (NFULL + 1) + cp]], rows[b], gsem[b]
            )

        def wait_gather(seq, b, cp=cp):
            pltpu.make_async_copy(
                table_hbm.at[idx_v.at[seq * (NFULL + 1) + cp]], rows[b], gsem[b]
            ).wait()

        def wait_scatter(b):
            pltpu.make_async_copy(
                rows[b], out_hbm.at[0, pl.ds(0, CS), :], ssem[b]
            ).wait()

        issue_gather(0, 0)

        def pair_body(s0, carry, cp=cp):
            for b in range(2):
                seq = s0 * 2 + b
                ob = 1 - b

                def prefetch(seq=seq, b=b, ob=ob):
                    if b == 0:
                        @pl.when(seq >= 1)
                        def _():
                            wait_scatter(ob)
                    else:
                        wait_scatter(ob)
                    issue_gather(seq + 1, ob)

                pl.when(seq + 1 < BPW)(prefetch)
                wait_gather(seq, b)

                def add_row(r, cc, b=b):
                    for k in range(KPR):
                        sl = pl.ds(k * LANES, LANES)
                        rows[b][r, sl] = rows[b][r, sl] + pe_v[r, sl]
                    return cc

                lax.fori_loop(0, CS, add_row, None)
                pltpu.async_copy(
                    rows[b],
                    out_hbm.at[wid * BPW + seq, pl.ds(cp * CS, CS), :],
                    ssem[b],
                )
            return carry

        lax.fori_loop(0, BPW // 2, pair_body, None)
        wait_scatter(0)
        wait_scatter(1)

    pltpu.sync_copy(pe_hbm.at[pl.ds(NFULL * CS, CS), :], pe_v)

    def tail_body(seq, carry):
        pltpu.async_copy(table_hbm.at[tidx_v.at[seq]], rows0, gsem0).wait()

        def add_row(r, cc):
            for k in range(KPR):
                sl = pl.ds(k * LANES, LANES)
                tail_v[r, sl] = rows0[r, sl] + pe_v[r, sl]
            return cc

        lax.fori_loop(0, TAIL, add_row, None)
        pltpu.sync_copy(
            tail_v, out_hbm.at[wid * BPW + seq, pl.ds(NFULL * CS, TAIL), :]
        )
        return carry

    lax.fori_loop(0, BPW, tail_body, None)


def kernel(x, emb_table):
    pe = jnp.asarray(
        np.pad(_pe_matrix_np(D_MODEL, SEQ_LEN), ((0, SEQ_PAD - SEQ_LEN), (0, 0)))
    )
    xi = x.astype(jnp.int32)
    xc = jnp.pad(xi, ((0, 0), (0, SEQ_PAD - SEQ_LEN))).reshape(
        NW, BPW * (NFULL + 1), CS
    )
    xt = jnp.pad(xi[:, NFULL * CS:], ((0, 0), (0, CS - TAIL))).reshape(
        NW, BPW, CS
    )
    return _emb_pe_kernel(xc, xt, emb_table, pe)


# speedup vs baseline: 1.9296x; 1.0019x over previous
import functools
import numpy as np
import jax
import jax.numpy as jnp
from jax import lax
from jax.experimental import pallas as pl
from jax.experimental.pallas import tpu as pltpu
from jax.experimental.pallas import tpu_sc as plsc

D_MODEL = 512
SEQ_LEN = 380
BATCH = 1024

NW = 32
BPW = BATCH // NW
CS = 32
NFULL = SEQ_LEN // CS
TAIL = SEQ_LEN - NFULL * CS
SEQ_PAD = (NFULL + 1) * CS
LANES = 16
KPR = D_MODEL // LANES


def _pe_matrix_np(d_model, seq_len):
    a, b = np.meshgrid(np.arange(d_model), np.arange(seq_len))
    pe_mat = b / 10000 ** (2 * (a // 2) / d_model)
    pe_mat[:, 0::2] = np.sin(pe_mat[:, 0::2])
    pe_mat[:, 1::2] = np.cos(pe_mat[:, 1::2])
    return pe_mat.astype(np.float32)


@functools.partial(
    pl.kernel,
    out_type=jax.ShapeDtypeStruct((BATCH, SEQ_LEN, D_MODEL), jnp.float32),
    mesh=plsc.VectorSubcoreMesh(core_axis_name="c", subcore_axis_name="s"),
    scratch_types=[
        pltpu.VMEM((BPW * (NFULL + 1), CS), jnp.int32),
        pltpu.VMEM((BPW, CS), jnp.int32),
        pltpu.VMEM((CS, D_MODEL), jnp.float32),
        pltpu.VMEM((CS, D_MODEL), jnp.float32),
        pltpu.VMEM((CS, D_MODEL), jnp.float32),
        pltpu.VMEM((TAIL, D_MODEL), jnp.float32),
        pltpu.SemaphoreType.DMA,
        pltpu.SemaphoreType.DMA,
        pltpu.SemaphoreType.DMA,
        pltpu.SemaphoreType.DMA,
    ],
)
def _emb_pe_kernel(xc_hbm, xt_hbm, table_hbm, pe_hbm, out_hbm,
                   idx_v, tidx_v, pe_v, rows0, rows1, tail_v,
                   gsem0, gsem1, ssem0, ssem1):
    rows = (rows0, rows1)
    gsem = (gsem0, gsem1)
    ssem = (ssem0, ssem1)

    wid = lax.axis_index("s") * 2 + lax.axis_index("c")
    pltpu.sync_copy(xc_hbm.at[wid], idx_v)
    pltpu.sync_copy(xt_hbm.at[wid], tidx_v)

    for cp in range(NFULL):
        pltpu.sync_copy(pe_hbm.at[pl.ds(cp * CS, CS), :], pe_v)

        def issue_gather(seq, b, cp=cp):
            pltpu.async_copy(
                table_hbm.at[idx_v.at[seq * (NFULL + 1) + cp]], rows[b], gsem[b]
            )

        def wait_gather(seq, b, cp=cp):
            pltpu.make_async_copy(
                table_hbm.at[idx_v.at[seq * (NFULL + 1) + cp]], rows[b], gsem[b]
            ).wait()

        def wait_scatter(b):
            pltpu.make_async_copy(
                rows[b], out_hbm.at[0, pl.ds(0, CS), :], ssem[b]
            ).wait()

        issue_gather(0, 0)

        def pair_body(s0, carry, cp=cp):
            for b in range(2):
                seq = s0 * 2 + b
                ob = 1 - b

                def prefetch(seq=seq, b=b, ob=ob):
                    if b == 0:
                        @pl.when(seq >= 1)
                        def _():
                            wait_scatter(ob)
                    else:
                        wait_scatter(ob)
                    issue_gather(seq + 1, ob)

                pl.when(seq + 1 < BPW)(prefetch)
                wait_gather(seq, b)

                def add_row(r, cc, b=b):
                    for k in range(KPR):
                        sl = pl.ds(k * LANES, LANES)
                        plsc.addupdate(rows[b].at[r, sl], pe_v[r, sl])
                    return cc

                lax.fori_loop(0, CS, add_row, None)
                pltpu.async_copy(
                    rows[b],
                    out_hbm.at[wid * BPW + seq, pl.ds(cp * CS, CS), :],
                    ssem[b],
                )
            return carry

        lax.fori_loop(0, BPW // 2, pair_body, None)
        wait_scatter(0)
        wait_scatter(1)

    pltpu.sync_copy(pe_hbm.at[pl.ds(NFULL * CS, CS), :], pe_v)

    def tail_body(seq, carry):
        pltpu.async_copy(table_hbm.at[tidx_v.at[seq]], rows0, gsem0).wait()

        def add_row(r, cc):
            for k in range(KPR):
                sl = pl.ds(k * LANES, LANES)
                tail_v[r, sl] = rows0[r, sl] + pe_v[r, sl]
            return cc

        lax.fori_loop(0, TAIL, add_row, None)
        pltpu.sync_copy(
            tail_v, out_hbm.at[wid * BPW + seq, pl.ds(NFULL * CS, TAIL), :]
        )
        return carry

    lax.fori_loop(0, BPW, tail_body, None)


def kernel(x, emb_table):
    pe = jnp.asarray(
        np.pad(_pe_matrix_np(D_MODEL, SEQ_LEN), ((0, SEQ_PAD - SEQ_LEN), (0, 0)))
    )
    xi = x.astype(jnp.int32)
    xc = jnp.pad(xi, ((0, 0), (0, SEQ_PAD - SEQ_LEN))).reshape(
        NW, BPW * (NFULL + 1), CS
    )
    xt = jnp.pad(xi[:, NFULL * CS:], ((0, 0), (0, CS - TAIL))).reshape(
        NW, BPW, CS
    )
    return _emb_pe_kernel(xc, xt, emb_table, pe)


# speedup vs baseline: 2.0686x; 1.0720x over previous
import functools
import numpy as np
import jax
import jax.numpy as jnp
from jax import lax
from jax.experimental import pallas as pl
from jax.experimental.pallas import tpu as pltpu
from jax.experimental.pallas import tpu_sc as plsc

D_MODEL = 512
SEQ_LEN = 380
BATCH = 1024

NW = 32
BPW = BATCH // NW
CS = 32
NFULL = SEQ_LEN // CS
TAIL = SEQ_LEN - NFULL * CS
NCHUNK = NFULL + 1
SEQ_PAD = NCHUNK * CS
LANES = 16
KPR = D_MODEL // LANES
GPC = BPW // 4


def _pe_matrix_np(d_model, seq_len):
    a, b = np.meshgrid(np.arange(d_model), np.arange(seq_len))
    pe_mat = b / 10000 ** (2 * (a // 2) / d_model)
    pe_mat[:, 0::2] = np.sin(pe_mat[:, 0::2])
    pe_mat[:, 1::2] = np.cos(pe_mat[:, 1::2])
    return pe_mat.astype(np.float32)


@functools.partial(
    pl.kernel,
    out_type=jax.ShapeDtypeStruct((BATCH, SEQ_LEN, D_MODEL), jnp.float32),
    mesh=plsc.VectorSubcoreMesh(core_axis_name="c", subcore_axis_name="s"),
    scratch_types=[
        pltpu.VMEM((NCHUNK * GPC, 4 * CS), jnp.int32),
        pltpu.VMEM((CS, D_MODEL), jnp.float32),
        pltpu.VMEM((CS, D_MODEL), jnp.float32),
        pltpu.VMEM((CS, D_MODEL), jnp.float32),
        pltpu.VMEM((CS, D_MODEL), jnp.float32),
        pltpu.VMEM((CS, D_MODEL), jnp.float32),
        pltpu.VMEM((TAIL, D_MODEL), jnp.float32),
        pltpu.SemaphoreType.DMA,
        pltpu.SemaphoreType.DMA,
        pltpu.SemaphoreType.DMA,
        pltpu.SemaphoreType.DMA,
        pltpu.SemaphoreType.DMA,
        pltpu.SemaphoreType.DMA,
        pltpu.SemaphoreType.DMA,
        pltpu.SemaphoreType.DMA,
    ],
)
def _emb_pe_kernel(xc_hbm, table_hbm, pe_hbm, out_hbm,
                   idx_v, pe_v, rows0, rows1, rows2, rows3, tail_v,
                   gsem0, gsem1, gsem2, gsem3, ssem0, ssem1, ssem2, ssem3):
    rows = (rows0, rows1, rows2, rows3)
    gsem = (gsem0, gsem1, gsem2, gsem3)
    ssem = (ssem0, ssem1, ssem2, ssem3)

    wid = lax.axis_index("s") * 2 + lax.axis_index("c")
    pltpu.sync_copy(xc_hbm.at[wid], idx_v)

    def idx_list(cp, q, jj):
        return idx_v.at[cp * GPC + q, pl.ds(jj * CS, CS)]

    def issue_gather(cp, q, jj, b):
        pltpu.async_copy(table_hbm.at[idx_list(cp, q, jj)], rows[b], gsem[b])

    def wait_gather(cp, q, jj, b):
        pltpu.make_async_copy(
            table_hbm.at[idx_list(cp, q, jj)], rows[b], gsem[b]
        ).wait()

    def wait_scatter(b):
        pltpu.make_async_copy(
            rows[b], out_hbm.at[0, pl.ds(0, CS), :], ssem[b]
        ).wait()

    def chunk_body(cp, carry):
        base = pl.multiple_of(cp * CS, CS)
        pltpu.sync_copy(pe_hbm.at[pl.ds(base, CS), :], pe_v)

        issue_gather(cp, 0, 0, 0)
        issue_gather(cp, 0, 1, 1)

        def quad_body(q, carry2, cp=cp):
            for jj in range(4):
                seq = q * 4 + jj

                def prefetch(q=q, jj=jj, cp=cp):
                    nb = (jj + 2) % 4
                    nq = q if jj < 2 else q + 1
                    nj = (jj + 2) % 4

                    @pl.when(q * 4 + jj >= 2)
                    def _():
                        wait_scatter(nb)

                    issue_gather(cp, nq, nj, nb)

                pl.when(seq + 2 < BPW)(prefetch)
                wait_gather(cp, q, jj, jj)

                def add_row(r, cc, jj=jj):
                    for k in range(KPR):
                        sl = pl.ds(k * LANES, LANES)
                        plsc.addupdate(rows[jj].at[r, sl], pe_v[r, sl])
                    return cc

                lax.fori_loop(0, CS, add_row, None)
                pltpu.async_copy(
                    rows[jj],
                    out_hbm.at[wid * BPW + seq, pl.ds(base, CS), :],
                    ssem[jj],
                )
            return carry2

        lax.fori_loop(0, GPC, quad_body, None)
        for jj in range(4):
            wait_scatter(jj)
        return carry

    lax.fori_loop(0, NFULL, chunk_body, None)

    pltpu.sync_copy(pe_hbm.at[pl.ds(NFULL * CS, CS), :], pe_v)

    def tail_quad(q, carry):
        for jj in range(4):
            seq = q * 4 + jj
            pltpu.async_copy(
                table_hbm.at[idx_list(NFULL, q, jj)], rows0, gsem0
            )
            pltpu.make_async_copy(
                table_hbm.at[idx_list(NFULL, q, jj)], rows0, gsem0
            ).wait()

            def add_row(r, cc):
                for k in range(KPR):
                    sl = pl.ds(k * LANES, LANES)
                    tail_v[r, sl] = rows0[r, sl] + pe_v[r, sl]
                return cc

            lax.fori_loop(0, TAIL, add_row, None)
            pltpu.sync_copy(
                tail_v, out_hbm.at[wid * BPW + seq, pl.ds(NFULL * CS, TAIL), :]
            )
        return carry

    lax.fori_loop(0, GPC, tail_quad, None)


def kernel(x, emb_table):
    pe = jnp.asarray(
        np.pad(_pe_matrix_np(D_MODEL, SEQ_LEN), ((0, SEQ_PAD - SEQ_LEN), (0, 0)))
    )
    xi = x.astype(jnp.int32)
    xc = (
        jnp.pad(xi, ((0, 0), (0, SEQ_PAD - SEQ_LEN)))
        .reshape(NW, BPW, NCHUNK, CS)
        .transpose(0, 2, 1, 3)
        .reshape(NW, NCHUNK * GPC, 4 * CS)
    )
    return _emb_pe_kernel(xc, emb_table, pe)
